# baseline (speedup 1.0000x reference)
import functools
import math

import jax
import jax.numpy as jnp
from jax.experimental import pallas as pl
from jax.experimental.pallas import tpu as pltpu


def _vmem_gather_kernel(ids_ref, w_hbm, o_ref, wvmem, tsem, *, tile, scale):
    c = pl.program_id(0)
    t = pl.program_id(1)

    @pl.when(t == 0)
    def _():
        cp = pltpu.make_async_copy(w_hbm.at[:, pl.ds(c, 1), :], wvmem, tsem)
        cp.start()
        cp.wait()

    base = t * tile
    for mi in range(tile):
        idx = ids_ref[base + mi]
        o_ref[mi, 0] = wvmem[idx, 0] * scale


def kernel(ids, weight):
    V, D = weight.shape
    orig_shape = ids.shape
    flat = ids.reshape(-1).astype(jnp.int32)
    n = flat.shape[0]
    scale = float(math.sqrt(D))

    flat = jnp.clip(flat, 0, V - 1)

    cores = 2
    dh = D // cores
    tile = 256
    n_pad = ((n + tile - 1) // tile) * tile
    if n_pad != n:
        flat = jnp.concatenate([flat, jnp.zeros((n_pad - n,), jnp.int32)])
    nt = n_pad // tile

    w3 = weight.reshape(V, cores, dh)

    emb = functools.partial(_vmem_gather_kernel, tile=tile, scale=scale)
    out = pl.pallas_call(
        emb,
        out_shape=jax.ShapeDtypeStruct((n_pad, 1, D), weight.dtype),
        grid_spec=pltpu.PrefetchScalarGridSpec(
            num_scalar_prefetch=1,
            grid=(cores, nt),
            in_specs=[pl.BlockSpec(memory_space=pl.ANY)],
            out_specs=pl.BlockSpec(
                (tile, 1, dh), lambda c, t, ids_smem: (t, 0, c)
            ),
            scratch_shapes=[
                pltpu.VMEM((V, 1, dh), weight.dtype),
                pltpu.SemaphoreType.DMA,
            ],
        ),
        compiler_params=pltpu.CompilerParams(
            dimension_semantics=("parallel", "arbitrary"),
            vmem_limit_bytes=60 * 1024 * 1024,
        ),
    )(flat, w3)
    return out[:n].reshape(*orig_shape, D)
